# baseline (speedup 1.0000x reference)
import jax
import jax.numpy as jnp
from jax.experimental import pallas as pl
from jax.experimental.pallas import tpu as pltpu
from jax.experimental.pallas import tpu_sc as plsc

_W = 128
_TR = 12480


def _sc_gather(idx, pe, R, D):
    mesh = plsc.VectorSubcoreMesh(core_axis_name="c", subcore_axis_name="s")

    @pl.kernel(out_type=jax.ShapeDtypeStruct((R, D), jnp.float32), mesh=mesh)
    def gather_k(i_hbm, pe_hbm, g_hbm):
        def body(i_vmem, g_vmem):
            pltpu.sync_copy(pe_hbm.at[i_vmem.at[0]], g_vmem)

        pltpu.emit_pipeline(
            body,
            grid=(R // _W,),
            in_specs=[pl.BlockSpec((1, _W), lambda i: (0, i))],
            out_specs=[pl.BlockSpec((_W, D), lambda i: (i, 0))],
            core_axis_name=("c", "s"),
            dimension_semantics=(pltpu.PARALLEL,),
        )(i_hbm, g_hbm)

    return gather_k(idx, pe)


def _tc_add(x, g, R, D):
    def add_k(x_ref, g_ref, o_ref):
        o_ref[...] = x_ref[...] + g_ref[...]

    return pl.pallas_call(
        add_k,
        grid=(R // _TR,),
        in_specs=[
            pl.BlockSpec((_TR, D), lambda i: (i, 0)),
            pl.BlockSpec((_TR, D), lambda i: (i, 0)),
        ],
        out_specs=pl.BlockSpec((_TR, D), lambda i: (i, 0)),
        out_shape=jax.ShapeDtypeStruct((R, D), jnp.float32),
    )(x, g)


def kernel(input_emb, position, pe):
    B, N, L, D = input_emb.shape
    R = B * N * L

    @jax.jit
    def run(input_emb, position, pe):
        x = input_emb.transpose(1, 2, 0, 3).reshape(R, D)
        idx = position.transpose(1, 2, 0).reshape(1, R).astype(jnp.int32)
        g = _sc_gather(idx, pe, R, D)
        out = _tc_add(x, g, R, D)
        return out.reshape(N, L, B, D).transpose(2, 0, 1, 3)

    return run(input_emb, position, pe)
